# baseline (speedup 1.0000x reference)
import functools
import math

import jax
import jax.numpy as jnp
from jax import lax
from jax.experimental import pallas as pl
from jax.experimental.pallas import tpu as pltpu
from jax.experimental.pallas import tpu_sc as plsc

D_MODEL = 64
LANES = 16
NUM_CORES = 2
NUM_SUBCORES = 16
NUM_WORKERS = NUM_CORES * NUM_SUBCORES
SCALE = math.sqrt(D_MODEL)

B = 4096
L = 200
BT = B // 128
LT = L // 8
G = 2
N_ITEM = G * 128
PER_W = L // 2
OUT_LEN = B * L * D_MODEL
IDX_ALL = LT * G * 8 * 128


def _make_kernel():
    mesh = plsc.VectorSubcoreMesh(core_axis_name="c", subcore_axis_name="s")

    scratch = (
        [pltpu.VMEM((IDX_ALL,), jnp.int32)]
        + [pltpu.VMEM((N_ITEM, D_MODEL), jnp.float32) for _ in range(2)]
        + [pltpu.VMEM((N_ITEM * D_MODEL,), jnp.float32) for _ in range(2)]
        + [pltpu.VMEM((8 * 1088,), jnp.float32)]
        + [pltpu.SemaphoreType.DMA for _ in range(5)]
    )

    @functools.partial(
        pl.kernel,
        mesh=mesh,
        out_type=jax.ShapeDtypeStruct((OUT_LEN,), jnp.float32),
        scratch_types=scratch,
        compiler_params=pltpu.CompilerParams(
            use_tc_tiling_on_sc=False, needs_layout_passes=False),
    )
    def emb_kernel(x_hbm, table_hbm, out_hbm,
                   idx_all, rows0, rows1, st0, st1, sb,
                   isem, gsem0, gsem1, osem0, osem1):
        rows = (rows0, rows1)
        stage = (st0, st1)
        gsem = (gsem0, gsem1)
        osem = (osem0, osem1)

        wid = lax.axis_index("s") * NUM_CORES + lax.axis_index("c")
        base_l = wid // 16
        bt0 = (wid % 16) * G

        for lt in range(LT):
            for g in range(G):
                pltpu.async_copy(
                    x_hbm.at[pl.ds((lt * BT + bt0 + g) * 1024, 1024)],
                    idx_all.at[pl.ds((lt * G + g) * 1024, 1024)], isem)
        for _ in range(LT * G):
            pltpu.make_async_copy(
                x_hbm.at[pl.ds(0, 1024)], idx_all.at[pl.ds(0, 1024)],
                isem).wait()

        def item_l(k):
            return base_l + 2 * k

        def gather_start(k, s):
            l = item_l(k)
            lt = l // 8
            r = l - lt * 8
            for g in range(G):
                pltpu.async_copy(
                    table_hbm.at[idx_all.at[
                        pl.ds(((lt * G + g) * 8 + r) * 128, 128)]],
                    rows[s].at[pl.ds(g * 128, 128)], gsem[s])

        def gather_wait(k, s):
            l = item_l(k)
            lt = l // 8
            r = l - lt * 8
            for g in range(G):
                pltpu.make_async_copy(
                    table_hbm.at[idx_all.at[
                        pl.ds(((lt * G + g) * 8 + r) * 128, 128)]],
                    rows[s].at[pl.ds(g * 128, 128)], gsem[s]).wait()

        def out_off(l, dt):
            return ((l * 8 + dt) * BT + bt0) * 1024

        def store_start(k, s):
            l = item_l(k)
            for dt in range(8):
                pltpu.async_copy(
                    stage[s].at[pl.ds(dt * G * 1024, G * 1024)],
                    out_hbm.at[pl.ds(out_off(l, dt), G * 1024)],
                    osem[s])

        def store_wait(k, s):
            l = item_l(k)
            for dt in range(8):
                pltpu.make_async_copy(
                    stage[s].at[pl.ds(dt * G * 1024, G * 1024)],
                    out_hbm.at[pl.ds(out_off(l, dt), G * 1024)],
                    osem[s]).wait()

        def transpose_scale(s):
            r = rows[s]
            st = stage[s]
            iota17 = lax.broadcasted_iota(jnp.int32, (LANES,), 0) * 17

            @plsc.parallel_loop(0, 8, unroll=2)
            def _(i):
                base = i * 1088
                for half in range(2):
                    a = i * 2 + half
                    dyn = (a // 8) * 1024 + (a % 8) * 16
                    row0 = a * 16
                    for db in range(4):
                        for rr in range(16):
                            sb[pl.ds(base + db * 272 + rr * 17, 16)] = \
                                r[row0 + rr, pl.ds(db * 16, 16)]
                    for db in range(4):
                        for cc in range(16):
                            d = db * 16 + cc
                            dt, dr = d // 8, d % 8
                            v = plsc.load_gather(
                                sb, [iota17 + (base + db * 272 + cc)])
                            st[pl.ds(dyn + dt * 2048 + dr * 128, 16)] = \
                                v * SCALE

        gather_start(0, 0)

        def step(kk, carry):
            gather_start(2 * kk + 1, 1)
            gather_wait(2 * kk, 0)

            @pl.when(kk > 0)
            def _():
                store_wait(2 * kk - 2, 0)

            transpose_scale(0)
            store_start(2 * kk, 0)

            @pl.when(kk < PER_W // 2 - 1)
            def _():
                gather_start(2 * kk + 2, 0)

            gather_wait(2 * kk + 1, 1)

            @pl.when(kk > 0)
            def _():
                store_wait(2 * kk - 1, 1)

            transpose_scale(1)
            store_start(2 * kk + 1, 1)
            return carry

        lax.fori_loop(0, PER_W // 2, step, 0)

        store_wait(PER_W - 2, 0)
        store_wait(PER_W - 1, 1)

    return emb_kernel


@jax.jit
def kernel(x, table):
    x1d = (x.astype(jnp.int32).T
           .reshape(LT, 8, BT, 128).transpose(0, 2, 1, 3).reshape(-1))
    o1d = _make_kernel()(x1d, table)
    return (o1d.reshape(L, 8, BT, 8, 128)
            .transpose(2, 4, 0, 1, 3).reshape(B, L, D_MODEL))


# speedup vs baseline: 1.2294x; 1.2294x over previous
import functools
import math

import jax
import jax.numpy as jnp
from jax import lax
from jax.experimental import pallas as pl
from jax.experimental.pallas import tpu as pltpu
from jax.experimental.pallas import tpu_sc as plsc

D_MODEL = 64
LANES = 16
NUM_CORES = 2
NUM_SUBCORES = 16
NUM_WORKERS = NUM_CORES * NUM_SUBCORES
SCALE = math.sqrt(D_MODEL)

B = 4096
L = 200
BT = B // 128
LT = L // 8
G = 2
N_ITEM = G * 128
PER_W = L // 2
OUT_LEN = B * L * D_MODEL
IDX_ALL = LT * G * 8 * 128


def _make_kernel():
    mesh = plsc.VectorSubcoreMesh(core_axis_name="c", subcore_axis_name="s")

    scratch = (
        [pltpu.VMEM((IDX_ALL,), jnp.int32)]
        + [pltpu.VMEM((N_ITEM, D_MODEL), jnp.float32) for _ in range(2)]
        + [pltpu.VMEM((N_ITEM * D_MODEL,), jnp.float32) for _ in range(2)]
        + [pltpu.VMEM((8 * 1088,), jnp.float32)]
        + [pltpu.SemaphoreType.DMA for _ in range(5)]
    )

    @functools.partial(
        pl.kernel,
        mesh=mesh,
        out_type=jax.ShapeDtypeStruct((OUT_LEN,), jnp.float32),
        scratch_types=scratch,
        compiler_params=pltpu.CompilerParams(
            use_tc_tiling_on_sc=False, needs_layout_passes=False),
    )
    def emb_kernel(x_hbm, table_hbm, out_hbm,
                   idx_all, rows0, rows1, st0, st1, sb,
                   isem, gsem0, gsem1, osem0, osem1):
        rows = (rows0, rows1)
        stage = (st0, st1)
        gsem = (gsem0, gsem1)
        osem = (osem0, osem1)

        wid = lax.axis_index("s") * NUM_CORES + lax.axis_index("c")
        base_l = wid // 16
        bt0 = (wid % 16) * G

        for lt in range(LT):
            for g in range(G):
                pltpu.async_copy(
                    x_hbm.at[pl.ds((lt * BT + bt0 + g) * 1024, 1024)],
                    idx_all.at[pl.ds((lt * G + g) * 1024, 1024)], isem)
        for _ in range(LT * G):
            pltpu.make_async_copy(
                x_hbm.at[pl.ds(0, 1024)], idx_all.at[pl.ds(0, 1024)],
                isem).wait()

        def item_l(k):
            return base_l + 2 * k

        def gather_start(k, s):
            l = item_l(k)
            lt = l // 8
            r = l - lt * 8
            for g in range(G):
                pltpu.async_copy(
                    table_hbm.at[idx_all.at[
                        pl.ds(((lt * G + g) * 8 + r) * 128, 128)]],
                    rows[s].at[pl.ds(g * 128, 128)], gsem[s])

        def gather_wait(k, s):
            l = item_l(k)
            lt = l // 8
            r = l - lt * 8
            for g in range(G):
                pltpu.make_async_copy(
                    table_hbm.at[idx_all.at[
                        pl.ds(((lt * G + g) * 8 + r) * 128, 128)]],
                    rows[s].at[pl.ds(g * 128, 128)], gsem[s]).wait()

        def out_off(l, dt):
            return ((l * 8 + dt) * BT + bt0) * 1024

        def store_start(k, s):
            l = item_l(k)
            for dt in range(8):
                pltpu.async_copy(
                    stage[s].at[pl.ds(dt * G * 1024, G * 1024)],
                    out_hbm.at[pl.ds(out_off(l, dt), G * 1024)],
                    osem[s])

        def store_wait(k, s):
            l = item_l(k)
            for dt in range(8):
                pltpu.make_async_copy(
                    stage[s].at[pl.ds(dt * G * 1024, G * 1024)],
                    out_hbm.at[pl.ds(out_off(l, dt), G * 1024)],
                    osem[s]).wait()

        def transpose_scale(s):
            r = rows[s]
            st = stage[s]
            iota17 = lax.broadcasted_iota(jnp.int32, (LANES,), 0) * 17

            def p1_ops(a, base):
                row0 = a * 16
                return [(base + db * 272 + rr * 17, row0 + rr, db * 16)
                        for db in range(4) for rr in range(16)]

            def p2_ops(a, base):
                dyn = (a // 8) * 1024 + (a % 8) * 16
                out = []
                for db in range(4):
                    for cc in range(16):
                        d = db * 16 + cc
                        out.append((base + db * 272 + cc,
                                    dyn + (d // 8) * 2048 + (d % 8) * 128))
                return out

            def emit(p1, p2):
                n = max(len(p1), len(p2))
                for i in range(n):
                    if i < len(p1):
                        boff, row, col = p1[i]
                        sb[pl.ds(boff, 16)] = r[row, pl.ds(col, 16)]
                    if i < len(p2):
                        goff, soff = p2[i]
                        v = plsc.load_gather(sb, [iota17 + goff])
                        st[pl.ds(soff, 16)] = v * SCALE

            A = 0
            BB = 1088

            def body(a2, c):
                a = a2 * 2
                emit(p1_ops(a + 1, BB), p2_ops(a, A))
                last = a2 == 7

                def nxt(cc):
                    emit(p1_ops(a + 2, A), p2_ops(a + 1, BB))
                    return cc

                def fin(cc):
                    emit([], p2_ops(a + 1, BB))
                    return cc

                lax.cond(a2 < 7, nxt, fin, c)
                return c

            emit(p1_ops(0, A), [])
            lax.fori_loop(0, 8, body, 0)

        gather_start(0, 0)

        def step(kk, carry):
            gather_start(2 * kk + 1, 1)
            gather_wait(2 * kk, 0)

            @pl.when(kk > 0)
            def _():
                store_wait(2 * kk - 2, 0)

            transpose_scale(0)
            store_start(2 * kk, 0)

            @pl.when(kk < PER_W // 2 - 1)
            def _():
                gather_start(2 * kk + 2, 0)

            gather_wait(2 * kk + 1, 1)

            @pl.when(kk > 0)
            def _():
                store_wait(2 * kk - 1, 1)

            transpose_scale(1)
            store_start(2 * kk + 1, 1)
            return carry

        lax.fori_loop(0, PER_W // 2, step, 0)

        store_wait(PER_W - 2, 0)
        store_wait(PER_W - 1, 1)

    return emb_kernel


@jax.jit
def kernel(x, table):
    x1d = (x.astype(jnp.int32).T
           .reshape(LT, 8, BT, 128).transpose(0, 2, 1, 3).reshape(-1))
    o1d = _make_kernel()(x1d, table)
    return (o1d.reshape(L, 8, BT, 8, 128)
            .transpose(2, 4, 0, 1, 3).reshape(B, L, D_MODEL))


# speedup vs baseline: 1.3843x; 1.1260x over previous
import functools
import math

import jax
import jax.numpy as jnp
from jax import lax
from jax.experimental import pallas as pl
from jax.experimental.pallas import tpu as pltpu
from jax.experimental.pallas import tpu_sc as plsc

D_MODEL = 64
LANES = 16
NUM_CORES = 2
NUM_SUBCORES = 16
NUM_WORKERS = NUM_CORES * NUM_SUBCORES
SCALE = math.sqrt(D_MODEL)

B = 4096
L = 200
BT = B // 128
LT = L // 8
G = 2
N_ITEM = G * 128
PER_W = L // 2
OUT_LEN = B * L * D_MODEL
IDX_ALL = LT * G * 8 * 128


def _make_kernel():
    mesh = plsc.VectorSubcoreMesh(core_axis_name="c", subcore_axis_name="s")

    scratch = (
        [pltpu.VMEM((IDX_ALL,), jnp.int32)]
        + [pltpu.VMEM((N_ITEM, D_MODEL), jnp.float32) for _ in range(2)]
        + [pltpu.VMEM((N_ITEM * D_MODEL,), jnp.float32) for _ in range(2)]
        + [pltpu.VMEM((4 * 16 * 17,), jnp.float32)]
        + [pltpu.SemaphoreType.DMA for _ in range(5)]
    )

    @functools.partial(
        pl.kernel,
        mesh=mesh,
        out_type=jax.ShapeDtypeStruct((OUT_LEN,), jnp.float32),
        scratch_types=scratch,
        compiler_params=pltpu.CompilerParams(
            use_tc_tiling_on_sc=False, needs_layout_passes=False,
            skip_device_barrier=True),
    )
    def emb_kernel(x_hbm, table_hbm, out_hbm,
                   idx_all, rows0, rows1, st0, st1, sb,
                   isem, gsem0, gsem1, osem0, osem1):
        rows = (rows0, rows1)
        stage = (st0, st1)
        gsem = (gsem0, gsem1)
        osem = (osem0, osem1)

        wid = lax.axis_index("s") * NUM_CORES + lax.axis_index("c")
        base_l = wid // 16
        bt0 = (wid % 16) * G

        for lt in range(LT):
            for g in range(G):
                pltpu.async_copy(
                    x_hbm.at[pl.ds((lt * BT + bt0 + g) * 1024, 1024)],
                    idx_all.at[pl.ds((lt * G + g) * 1024, 1024)], isem)
        for _ in range(LT * G):
            pltpu.make_async_copy(
                x_hbm.at[pl.ds(0, 1024)], idx_all.at[pl.ds(0, 1024)],
                isem).wait()

        def item_l(k):
            return base_l + 2 * k

        def gather_start(k, s):
            l = item_l(k)
            lt = l // 8
            r = l - lt * 8
            for g in range(G):
                pltpu.async_copy(
                    table_hbm.at[idx_all.at[
                        pl.ds(((lt * G + g) * 8 + r) * 128, 128)]],
                    rows[s].at[pl.ds(g * 128, 128)], gsem[s])

        def gather_wait(k, s):
            l = item_l(k)
            lt = l // 8
            r = l - lt * 8
            for g in range(G):
                pltpu.make_async_copy(
                    table_hbm.at[idx_all.at[
                        pl.ds(((lt * G + g) * 8 + r) * 128, 128)]],
                    rows[s].at[pl.ds(g * 128, 128)], gsem[s]).wait()

        def out_off(l, dt):
            return ((l * 8 + dt) * BT + bt0) * 1024

        def store_start(k, s):
            l = item_l(k)
            for dt in range(8):
                pltpu.async_copy(
                    stage[s].at[pl.ds(dt * G * 1024, G * 1024)],
                    out_hbm.at[pl.ds(out_off(l, dt), G * 1024)],
                    osem[s])

        def store_wait(k, s):
            l = item_l(k)
            for dt in range(8):
                pltpu.make_async_copy(
                    stage[s].at[pl.ds(dt * G * 1024, G * 1024)],
                    out_hbm.at[pl.ds(out_off(l, dt), G * 1024)],
                    osem[s]).wait()

        def transpose_scale(s):
            r = rows[s]
            st = stage[s]
            iota17 = lax.broadcasted_iota(jnp.int32, (LANES,), 0) * 17

            def body(a, c):
                dyn = (a // 8) * 1024 + (a % 8) * 16
                row0 = a * 16
                for db in range(4):
                    for rr in range(16):
                        sb[pl.ds(db * 272 + rr * 17, 16)] = \
                            r[row0 + rr, pl.ds(db * 16, 16)]
                for db in range(4):
                    for cc in range(16):
                        d = db * 16 + cc
                        dt, dr = d // 8, d % 8
                        v = plsc.load_gather(
                            sb, [iota17 + (db * 272 + cc)])
                        st[pl.ds(dyn + dt * 2048 + dr * 128, 16)] = v * SCALE
                return c

            lax.fori_loop(0, 16, body, 0)

        gather_start(0, 0)

        def step(kk, carry):
            gather_start(2 * kk + 1, 1)
            gather_wait(2 * kk, 0)

            @pl.when(kk > 0)
            def _():
                store_wait(2 * kk - 2, 0)

            transpose_scale(0)
            store_start(2 * kk, 0)

            @pl.when(kk < PER_W // 2 - 1)
            def _():
                gather_start(2 * kk + 2, 0)

            gather_wait(2 * kk + 1, 1)

            @pl.when(kk > 0)
            def _():
                store_wait(2 * kk - 1, 1)

            transpose_scale(1)
            store_start(2 * kk + 1, 1)
            return carry

        lax.fori_loop(0, PER_W // 2, step, 0)

        store_wait(PER_W - 2, 0)
        store_wait(PER_W - 1, 1)

    return emb_kernel


@jax.jit
def kernel(x, table):
    x1d = (x.astype(jnp.int32).T
           .reshape(LT, 8, BT, 128).transpose(0, 2, 1, 3).reshape(-1))
    o1d = _make_kernel()(x1d, table)
    return (o1d.reshape(L, 8, BT, 8, 128)
            .transpose(2, 4, 0, 1, 3).reshape(B, L, D_MODEL))


# speedup vs baseline: 1.6096x; 1.1627x over previous
import functools
import math

import jax
import jax.numpy as jnp
from jax import lax
from jax.experimental import pallas as pl
from jax.experimental.pallas import tpu as pltpu
from jax.experimental.pallas import tpu_sc as plsc

D_MODEL = 64
LANES = 16
NUM_CORES = 2
NUM_SUBCORES = 16
NUM_WORKERS = NUM_CORES * NUM_SUBCORES
SCALE = math.sqrt(D_MODEL)

CHUNK = 512
NSLOT = 3


def _make_kernel(n_idx):
    assert n_idx % (NUM_WORKERS * CHUNK) == 0
    per_worker = n_idx // NUM_WORKERS
    n_chunks = per_worker // CHUNK
    mesh = plsc.VectorSubcoreMesh(core_axis_name="c", subcore_axis_name="s")

    scratch = (
        [pltpu.VMEM((CHUNK,), jnp.int32) for _ in range(NSLOT)]
        + [pltpu.VMEM((CHUNK, D_MODEL), jnp.float32) for _ in range(NSLOT)]
        + [pltpu.SemaphoreType.DMA for _ in range(3 * NSLOT)]
    )

    @functools.partial(
        pl.kernel,
        mesh=mesh,
        out_type=jax.ShapeDtypeStruct((n_idx, D_MODEL), jnp.float32),
        scratch_types=scratch,
        compiler_params=pltpu.CompilerParams(use_tc_tiling_on_sc=False),
    )
    def emb_kernel(x_hbm, table_hbm, out_hbm, *s):
        idx = s[0:NSLOT]
        rows = s[NSLOT:2 * NSLOT]
        isem = s[2 * NSLOT:3 * NSLOT]
        gsem = s[3 * NSLOT:4 * NSLOT]
        osem = s[4 * NSLOT:5 * NSLOT]

        wid = lax.axis_index("s") * NUM_CORES + lax.axis_index("c")
        base = wid * per_worker

        def idx_start(i):
            pltpu.async_copy(
                x_hbm.at[pl.ds(base + i * CHUNK, CHUNK)], idx[i % NSLOT],
                isem[i % NSLOT])

        def idx_wait(i):
            pltpu.make_async_copy(
                x_hbm.at[pl.ds(base + i * CHUNK, CHUNK)], idx[i % NSLOT],
                isem[i % NSLOT]).wait()

        def gather_start(i):
            pltpu.async_copy(
                table_hbm.at[idx[i % NSLOT]], rows[i % NSLOT], gsem[i % NSLOT])

        def gather_wait(i):
            pltpu.make_async_copy(
                table_hbm.at[idx[i % NSLOT]], rows[i % NSLOT],
                gsem[i % NSLOT]).wait()

        def store_start(i):
            pltpu.async_copy(
                rows[i % NSLOT], out_hbm.at[pl.ds(base + i * CHUNK, CHUNK)],
                osem[i % NSLOT])

        def store_wait(i):
            pltpu.make_async_copy(
                rows[i % NSLOT], out_hbm.at[pl.ds(base + i * CHUNK, CHUNK)],
                osem[i % NSLOT]).wait()

        def scale(i):
            r = rows[i % NSLOT]

            def scale_body(t, c):
                row = t * 4
                for u in range(4):
                    for j in range(D_MODEL // LANES):
                        sl = pl.ds(j * LANES, LANES)
                        r[row + u, sl] = r[row + u, sl] * SCALE
                return c

            lax.fori_loop(0, CHUNK // 4, scale_body, 0)

        for i in range(min(NSLOT, n_chunks)):
            idx_start(i)
        idx_wait(0)
        gather_start(0)

        for i in range(n_chunks):
            gather_wait(i)
            if i + NSLOT < n_chunks:
                idx_start(i + NSLOT)
            if i + 1 < n_chunks:
                if i - (NSLOT - 1) >= 0:
                    store_wait(i - (NSLOT - 1))
                idx_wait(i + 1)
                gather_start(i + 1)
            scale(i)
            store_start(i)

        for i in range(max(0, n_chunks - NSLOT), n_chunks):
            store_wait(i)

    return emb_kernel


@jax.jit
def kernel(x, table):
    b, l = x.shape
    x_flat = x.reshape((b * l,)).astype(jnp.int32)
    out = _make_kernel(b * l)(x_flat, table)
    return out.reshape((b, l, D_MODEL))
